# Initial kernel scaffold; baseline (speedup 1.0000x reference)
import jax
import jax.numpy as jnp
from jax.experimental import pallas as pl


def kernel(x, edge_index, edge_attr, batch, app_embedding, W1, b1, W2, b2, Wa, ba, Ws, bs, Wo, bo):
    raise NotImplementedError("write your pallas kernel here")


# speedup vs baseline: 10.1101x; 10.1101x over previous
import functools

import jax
import jax.numpy as jnp
from jax import lax
from jax.experimental import pallas as pl
from jax.experimental.pallas import tpu as pltpu
from jax.experimental.pallas import tpu_sc as plsc

N = 10000
E = 320000
B = 4
D = 128
H = 128
EMB = 64
ATT = 64

NC = 2
NS = 16
NW = NC * NS
EP_W = E // NW
K = 80
NCHUNK = EP_W // K
ZROWS = 125
RP_S = N // NS

_mesh = plsc.VectorSubcoreMesh(core_axis_name="c", subcore_axis_name="s")


@functools.partial(
    pl.kernel,
    out_type=jax.ShapeDtypeStruct((NW, N), jnp.float32),
    mesh=_mesh,
    scratch_types=[
        pltpu.VMEM((EP_W,), jnp.int32),
        pltpu.VMEM((EP_W,), jnp.float32),
        pltpu.VMEM((N,), jnp.float32),
    ],
    compiler_params=pltpu.CompilerParams(needs_layout_passes=False, use_tc_tiling_on_sc=False),
)
def _deg_kernel(dst_hbm, ew_hbm, degs_hbm, dst_v, ew_v, deg_v):
    wid = lax.axis_index("s") * NC + lax.axis_index("c")
    base = wid * EP_W

    def zero(i, carry):
        deg_v[pl.ds(i * 16, 16)] = jnp.zeros((16,), jnp.float32)
        return carry

    lax.fori_loop(0, N // 16, zero, 0)

    pltpu.sync_copy(dst_hbm.at[pl.ds(base, EP_W)], dst_v)
    pltpu.sync_copy(ew_hbm.at[pl.ds(base, EP_W)], ew_v)

    def body(j, carry):
        d16 = dst_v[pl.ds(j * 16, 16)]
        w16 = ew_v[pl.ds(j * 16, 16)]
        plsc.addupdate_scatter(deg_v, [d16], w16)
        return carry

    lax.fori_loop(0, EP_W // 16, body, 0)
    pltpu.sync_copy(deg_v, degs_hbm.at[wid])


@functools.partial(
    pl.kernel,
    out_type=jax.ShapeDtypeStruct((NC, N, H), jnp.float32),
    mesh=_mesh,
    scratch_types=[
        pltpu.VMEM((K,), jnp.int32),
        pltpu.VMEM((K,), jnp.int32),
        pltpu.VMEM((K,), jnp.float32),
        pltpu.VMEM((K, H), jnp.float32),
        pltpu.VMEM((ZROWS, H), jnp.float32),
        pltpu.VMEM_SHARED((N, H), jnp.float32),
        pltpu.SemaphoreType.DMA,
    ],
    compiler_params=pltpu.CompilerParams(needs_layout_passes=False, use_tc_tiling_on_sc=False),
)
def _agg_kernel(src_hbm, dst_hbm, ew_hbm, ys_hbm, parts_hbm,
                src_v, dst_v, ew_v, rows_v, zbuf_v, acc_sh, sem):
    c = lax.axis_index("c")
    s = lax.axis_index("s")
    wid = s * NC + c

    def zero(i, carry):
        for v in range(H // 16):
            zbuf_v[i, pl.ds(v * 16, 16)] = jnp.zeros((16,), jnp.float32)
        return carry

    lax.fori_loop(0, ZROWS, zero, 0)
    for k in range(RP_S // ZROWS):
        pltpu.sync_copy(zbuf_v, acc_sh.at[pl.ds(s * RP_S + k * ZROWS, ZROWS)])
    plsc.subcore_barrier()

    ebase = wid * EP_W

    def chunk(cix, carry):
        base = ebase + cix * K
        pltpu.sync_copy(src_hbm.at[pl.ds(base, K)], src_v)
        pltpu.sync_copy(dst_hbm.at[pl.ds(base, K)], dst_v)
        pltpu.sync_copy(ew_hbm.at[pl.ds(base, K)], ew_v)
        pltpu.async_copy(ys_hbm.at[src_v], rows_v, sem).wait()

        def scale(j, inner):
            idx16 = jnp.zeros((16,), jnp.int32) + j
            bc = plsc.load_gather(ew_v, [idx16])
            for v in range(H // 16):
                rows_v[j, pl.ds(v * 16, 16)] = rows_v[j, pl.ds(v * 16, 16)] * bc
            return inner

        lax.fori_loop(0, K, scale, 0)
        pltpu.sync_copy(rows_v, acc_sh.at[dst_v], add=True)
        return carry

    lax.fori_loop(0, NCHUNK, chunk, 0)
    plsc.subcore_barrier()
    for k in range(RP_S // ZROWS):
        off = s * RP_S + k * ZROWS
        pltpu.sync_copy(acc_sh.at[pl.ds(off, ZROWS)],
                        parts_hbm.at[c, pl.ds(off, ZROWS)])


def _tcA_body(x_ref, w1_ref, degs_ref, ys_ref, dinv_ref):
    deg = jnp.sum(degs_ref[...], axis=0) + 1.0
    dinv = lax.rsqrt(deg)
    xw = lax.dot_general(x_ref[...], w1_ref[...], (((1,), (1,)), ((), ())),
                         preferred_element_type=jnp.float32)
    ys_ref[...] = xw * dinv[:, None]
    dinv_ref[...] = dinv


def _tcA(x, W1, degs):
    return pl.pallas_call(
        _tcA_body,
        out_shape=[jax.ShapeDtypeStruct((N, H), jnp.float32),
                   jax.ShapeDtypeStruct((N,), jnp.float32)],
    )(x, W1, degs)


def _tcC_body(parts_ref, ys1_ref, dinv_ref, b1_ref, w2_ref, ys2_ref):
    dinv = dinv_ref[...]
    acc = parts_ref[0] + parts_ref[1] + ys1_ref[...]
    h1 = jnp.maximum(acc * dinv[:, None] + b1_ref[...][None, :], 0.0)
    xw2 = lax.dot_general(h1, w2_ref[...], (((1,), (1,)), ((), ())),
                          preferred_element_type=jnp.float32)
    ys2_ref[...] = xw2 * dinv[:, None]


def _tcC(parts1, ys1, dinv, b1, W2):
    return pl.pallas_call(
        _tcC_body,
        out_shape=jax.ShapeDtypeStruct((N, H), jnp.float32),
    )(parts1, ys1, dinv, b1, W2)


def _tcD_body(parts_ref, ys2_ref, dinv_ref, b2_ref, batch_ref, emb_ref,
              wah_ref, wae_ref, ba_ref, ws_ref, bs_ref, wo_ref, bo_ref,
              out_ref):
    dinv = dinv_ref[...]
    acc = parts_ref[0] + parts_ref[1] + ys2_ref[...]
    h2 = jnp.maximum(acc * dinv[:, None] + b2_ref[...][None, :], 0.0)

    P = lax.dot_general(emb_ref[...], wae_ref[...], (((1,), (1,)), ((), ())),
                        preferred_element_type=jnp.float32)
    batch = batch_ref[...]
    oh = (batch[:, None] == lax.broadcasted_iota(jnp.int32, (1, B), 1)
          ).astype(jnp.float32)
    embp = lax.dot_general(oh, P, (((1,), (0,)), ((), ())),
                           preferred_element_type=jnp.float32)
    ah = lax.dot_general(h2, wah_ref[...], (((1,), (1,)), ((), ())),
                         preferred_element_type=jnp.float32) + embp \
        + ba_ref[...][None, :]
    sc = jnp.where(ah >= 0, ah, 0.2 * ah)
    logits = jnp.sum(sc * ws_ref[...], axis=1, keepdims=True) \
        + bs_ref[...][None, :]
    m = jnp.max(logits)
    e = jnp.exp(logits - m)
    aw = e / jnp.sum(e)
    w = aw * h2
    S = lax.dot_general(oh, w, (((0,), (0,)), ((), ())),
                        preferred_element_type=jnp.float32)
    cnt = jnp.sum(oh, axis=0)
    num = jnp.sum(S * wo_ref[...][0][None, :], axis=1) + cnt * bo_ref[...]
    out_ref[...] = num / jnp.maximum(cnt, 1.0)


def _tcD(parts2, ys2, dinv, b2, batch, app_embedding, Wah, Wae, ba, Ws, bs,
         Wo, bo):
    return pl.pallas_call(
        _tcD_body,
        out_shape=jax.ShapeDtypeStruct((B,), jnp.float32),
    )(parts2, ys2, dinv, b2, batch, app_embedding, Wah, Wae, ba, Ws, bs,
      Wo, bo)


def kernel(x, edge_index, edge_attr, batch, app_embedding, W1, b1, W2, b2,
           Wa, ba, Ws, bs, Wo, bo):
    src = edge_index[0]
    dst = edge_index[1]
    degs = _deg_kernel(dst, edge_attr)
    ys1, dinv = _tcA(x, W1, degs)
    parts1 = _agg_kernel(src, dst, edge_attr, ys1)
    ys2 = _tcC(parts1, ys1, dinv, b1, W2)
    parts2 = _agg_kernel(src, dst, edge_attr, ys2)
    Wah = Wa[:, :H]
    Wae = Wa[:, H:]
    return _tcD(parts2, ys2, dinv, b2, batch, app_embedding, Wah, Wae,
                ba, Ws, bs, Wo, bo)


# speedup vs baseline: 23.2044x; 2.2952x over previous
import functools

import jax
import jax.numpy as jnp
from jax import lax
from jax.experimental import pallas as pl
from jax.experimental.pallas import tpu as pltpu
from jax.experimental.pallas import tpu_sc as plsc

N = 10000
E = 320000
B = 4
D = 128
H = 128
EMB = 64
ATT = 64

NC = 2
NS = 16
NW = NC * NS
EP_W = E // NW
K = 80
NCHUNK = EP_W // K
ZROWS = 125
RP_S = N // NS

_mesh = plsc.VectorSubcoreMesh(core_axis_name="c", subcore_axis_name="s")


@functools.partial(
    pl.kernel,
    out_type=jax.ShapeDtypeStruct((NW, N), jnp.float32),
    mesh=_mesh,
    scratch_types=[
        pltpu.VMEM((EP_W,), jnp.int32),
        pltpu.VMEM((EP_W,), jnp.float32),
        pltpu.VMEM((N,), jnp.float32),
    ],
    compiler_params=pltpu.CompilerParams(needs_layout_passes=False, use_tc_tiling_on_sc=False),
)
def _deg_kernel(dst_hbm, ew_hbm, degs_hbm, dst_v, ew_v, deg_v):
    wid = lax.axis_index("s") * NC + lax.axis_index("c")
    base = wid * EP_W

    def zero(i, carry):
        deg_v[pl.ds(i * 16, 16)] = jnp.zeros((16,), jnp.float32)
        return carry

    lax.fori_loop(0, N // 16, zero, 0)

    pltpu.sync_copy(dst_hbm.at[pl.ds(base, EP_W)], dst_v)
    pltpu.sync_copy(ew_hbm.at[pl.ds(base, EP_W)], ew_v)

    def body(j, carry):
        d16 = dst_v[pl.ds(j * 16, 16)]
        w16 = ew_v[pl.ds(j * 16, 16)]
        plsc.addupdate_scatter(deg_v, [d16], w16)
        return carry

    lax.fori_loop(0, EP_W // 16, body, 0)
    pltpu.sync_copy(deg_v, degs_hbm.at[wid])


@functools.partial(
    pl.kernel,
    out_type=jax.ShapeDtypeStruct((NC, N, H), jnp.float32),
    mesh=_mesh,
    scratch_types=[
        pltpu.VMEM((K,), jnp.int32),
        pltpu.VMEM((K,), jnp.int32),
        pltpu.VMEM((K,), jnp.int32),
        pltpu.VMEM((K,), jnp.int32),
        pltpu.VMEM((K,), jnp.float32),
        pltpu.VMEM((K,), jnp.float32),
        pltpu.VMEM((K, H), jnp.float32),
        pltpu.VMEM((K, H), jnp.float32),
        pltpu.VMEM((ZROWS, H), jnp.float32),
        pltpu.VMEM_SHARED((N, H), jnp.float32),
        pltpu.SemaphoreType.DMA,
        pltpu.SemaphoreType.DMA,
        pltpu.SemaphoreType.DMA,
        pltpu.SemaphoreType.DMA,
    ],
    compiler_params=pltpu.CompilerParams(needs_layout_passes=False, use_tc_tiling_on_sc=False),
)
def _agg_kernel(src_hbm, dst_hbm, ew_hbm, ys_hbm, parts_hbm,
                src0, src1, dst0, dst1, ew0, ew1, rows0, rows1,
                zbuf_v, acc_sh, isem0, isem1, gsem0, gsem1):
    c = lax.axis_index("c")
    s = lax.axis_index("s")
    wid = s * NC + c

    srcs = (src0, src1)
    dsts = (dst0, dst1)
    ews = (ew0, ew1)
    rows = (rows0, rows1)
    isem = (isem0, isem1)
    gsem = (gsem0, gsem1)

    def zero(i, carry):
        for v in range(H // 16):
            zbuf_v[i, pl.ds(v * 16, 16)] = jnp.zeros((16,), jnp.float32)
        return carry

    lax.fori_loop(0, ZROWS, zero, 0)
    for k in range(RP_S // ZROWS):
        pltpu.sync_copy(zbuf_v, acc_sh.at[pl.ds(s * RP_S + k * ZROWS, ZROWS)])
    plsc.subcore_barrier()

    ebase = wid * EP_W

    def idx_start(cix, b):
        base = ebase + cix * K
        pltpu.async_copy(src_hbm.at[pl.ds(base, K)], srcs[b], isem[b])
        pltpu.async_copy(dst_hbm.at[pl.ds(base, K)], dsts[b], isem[b])
        pltpu.async_copy(ew_hbm.at[pl.ds(base, K)], ews[b], isem[b])

    def idx_wait(b):
        pltpu.make_async_copy(src_hbm.at[pl.ds(0, K)], srcs[b], isem[b]).wait()
        pltpu.make_async_copy(dst_hbm.at[pl.ds(0, K)], dsts[b], isem[b]).wait()
        pltpu.make_async_copy(ew_hbm.at[pl.ds(0, K)], ews[b], isem[b]).wait()

    def gather_start(b):
        pltpu.async_copy(ys_hbm.at[srcs[b]], rows[b], gsem[b])

    def gather_wait(b):
        pltpu.make_async_copy(ys_hbm.at[srcs[b]], rows[b], gsem[b]).wait()

    def process(b):
        @plsc.parallel_loop(0, K, unroll=4)
        def scale(j):
            idx16 = jnp.zeros((16,), jnp.int32) + j
            bc = plsc.load_gather(ews[b], [idx16])
            for v in range(H // 16):
                rows[b][j, pl.ds(v * 16, 16)] = \
                    rows[b][j, pl.ds(v * 16, 16)] * bc

        pltpu.sync_copy(rows[b], acc_sh.at[dsts[b]], add=True)

    idx_start(0, 0)
    idx_wait(0)
    gather_start(0)
    idx_start(1, 1)

    def pair(cp, carry):
        c0 = 2 * cp
        idx_wait(1)
        gather_start(1)
        gather_wait(0)
        process(0)
        idx_start(c0 + 2, 0)
        idx_wait(0)
        gather_start(0)
        gather_wait(1)
        process(1)

        @pl.when(c0 + 3 < NCHUNK)
        def _():
            idx_start(c0 + 3, 1)

        return carry

    lax.fori_loop(0, (NCHUNK - 1) // 2, pair, 0)
    gather_wait(0)
    process(0)

    plsc.subcore_barrier()
    for k in range(RP_S // ZROWS):
        off = s * RP_S + k * ZROWS
        pltpu.sync_copy(acc_sh.at[pl.ds(off, ZROWS)],
                        parts_hbm.at[c, pl.ds(off, ZROWS)])


def _tcA_body(x_ref, w1_ref, degs_ref, ys_ref, dinv_ref):
    deg = jnp.sum(degs_ref[...], axis=0) + 1.0
    dinv = lax.rsqrt(deg)
    xw = lax.dot_general(x_ref[...], w1_ref[...], (((1,), (1,)), ((), ())),
                         preferred_element_type=jnp.float32)
    ys_ref[...] = xw * dinv[:, None]
    dinv_ref[...] = dinv


def _tcA(x, W1, degs):
    return pl.pallas_call(
        _tcA_body,
        out_shape=[jax.ShapeDtypeStruct((N, H), jnp.float32),
                   jax.ShapeDtypeStruct((N,), jnp.float32)],
    )(x, W1, degs)


def _tcC_body(parts_ref, ys1_ref, dinv_ref, b1_ref, w2_ref, ys2_ref):
    dinv = dinv_ref[...]
    acc = parts_ref[0] + parts_ref[1] + ys1_ref[...]
    h1 = jnp.maximum(acc * dinv[:, None] + b1_ref[...][None, :], 0.0)
    xw2 = lax.dot_general(h1, w2_ref[...], (((1,), (1,)), ((), ())),
                          preferred_element_type=jnp.float32)
    ys2_ref[...] = xw2 * dinv[:, None]


def _tcC(parts1, ys1, dinv, b1, W2):
    return pl.pallas_call(
        _tcC_body,
        out_shape=jax.ShapeDtypeStruct((N, H), jnp.float32),
    )(parts1, ys1, dinv, b1, W2)


def _tcD_body(parts_ref, ys2_ref, dinv_ref, b2_ref, batch_ref, emb_ref,
              wah_ref, wae_ref, ba_ref, ws_ref, bs_ref, wo_ref, bo_ref,
              out_ref):
    dinv = dinv_ref[...]
    acc = parts_ref[0] + parts_ref[1] + ys2_ref[...]
    h2 = jnp.maximum(acc * dinv[:, None] + b2_ref[...][None, :], 0.0)

    P = lax.dot_general(emb_ref[...], wae_ref[...], (((1,), (1,)), ((), ())),
                        preferred_element_type=jnp.float32)
    batch = batch_ref[...]
    oh = (batch[:, None] == lax.broadcasted_iota(jnp.int32, (1, B), 1)
          ).astype(jnp.float32)
    embp = lax.dot_general(oh, P, (((1,), (0,)), ((), ())),
                           preferred_element_type=jnp.float32)
    ah = lax.dot_general(h2, wah_ref[...], (((1,), (1,)), ((), ())),
                         preferred_element_type=jnp.float32) + embp \
        + ba_ref[...][None, :]
    sc = jnp.where(ah >= 0, ah, 0.2 * ah)
    logits = jnp.sum(sc * ws_ref[...], axis=1, keepdims=True) \
        + bs_ref[...][None, :]
    m = jnp.max(logits)
    e = jnp.exp(logits - m)
    aw = e / jnp.sum(e)
    w = aw * h2
    S = lax.dot_general(oh, w, (((0,), (0,)), ((), ())),
                        preferred_element_type=jnp.float32)
    cnt = jnp.sum(oh, axis=0)
    num = jnp.sum(S * wo_ref[...][0][None, :], axis=1) + cnt * bo_ref[...]
    out_ref[...] = num / jnp.maximum(cnt, 1.0)


def _tcD(parts2, ys2, dinv, b2, batch, app_embedding, Wah, Wae, ba, Ws, bs,
         Wo, bo):
    return pl.pallas_call(
        _tcD_body,
        out_shape=jax.ShapeDtypeStruct((B,), jnp.float32),
    )(parts2, ys2, dinv, b2, batch, app_embedding, Wah, Wae, ba, Ws, bs,
      Wo, bo)


def kernel(x, edge_index, edge_attr, batch, app_embedding, W1, b1, W2, b2,
           Wa, ba, Ws, bs, Wo, bo):
    src = edge_index[0]
    dst = edge_index[1]
    degs = _deg_kernel(dst, edge_attr)
    ys1, dinv = _tcA(x, W1, degs)
    parts1 = _agg_kernel(src, dst, edge_attr, ys1)
    ys2 = _tcC(parts1, ys1, dinv, b1, W2)
    parts2 = _agg_kernel(src, dst, edge_attr, ys2)
    Wah = Wa[:, :H]
    Wae = Wa[:, H:]
    return _tcD(parts2, ys2, dinv, b2, batch, app_embedding, Wah, Wae,
                ba, Ws, bs, Wo, bo)


# speedup vs baseline: 26.2488x; 1.1312x over previous
import functools

import jax
import jax.numpy as jnp
from jax import lax
from jax.experimental import pallas as pl
from jax.experimental.pallas import tpu as pltpu
from jax.experimental.pallas import tpu_sc as plsc

N = 10000
E = 320000
B = 4
D = 128
H = 128
EMB = 64
ATT = 64

NC = 2
NS = 16
NW = NC * NS
EP_W = E // NW
K = 80
NCHUNK = EP_W // K
ZROWS = 125
RP_S = N // NS

_mesh = plsc.VectorSubcoreMesh(core_axis_name="c", subcore_axis_name="s")


@functools.partial(
    pl.kernel,
    out_type=jax.ShapeDtypeStruct((NW, N), jnp.float32),
    mesh=_mesh,
    scratch_types=[
        pltpu.VMEM((EP_W,), jnp.int32),
        pltpu.VMEM((EP_W,), jnp.float32),
        pltpu.VMEM((N,), jnp.float32),
    ],
    compiler_params=pltpu.CompilerParams(needs_layout_passes=False, use_tc_tiling_on_sc=False),
)
def _deg_kernel(dst_hbm, ew_hbm, degs_hbm, dst_v, ew_v, deg_v):
    wid = lax.axis_index("s") * NC + lax.axis_index("c")
    base = wid * EP_W

    def zero(i, carry):
        deg_v[pl.ds(i * 16, 16)] = jnp.zeros((16,), jnp.float32)
        return carry

    lax.fori_loop(0, N // 16, zero, 0)

    pltpu.sync_copy(dst_hbm.at[pl.ds(base, EP_W)], dst_v)
    pltpu.sync_copy(ew_hbm.at[pl.ds(base, EP_W)], ew_v)

    def body(j, carry):
        d16 = dst_v[pl.ds(j * 16, 16)]
        w16 = ew_v[pl.ds(j * 16, 16)]
        plsc.addupdate_scatter(deg_v, [d16], w16)
        return carry

    lax.fori_loop(0, EP_W // 16, body, 0)
    pltpu.sync_copy(deg_v, degs_hbm.at[wid])


@functools.partial(
    pl.kernel,
    out_type=jax.ShapeDtypeStruct((NC, N, H), jnp.float32),
    mesh=_mesh,
    scratch_types=[
        pltpu.VMEM((3, K), jnp.int32),
        pltpu.VMEM((3, K), jnp.int32),
        pltpu.VMEM((3, K), jnp.int32),
        pltpu.VMEM((3, K), jnp.int32),
        pltpu.VMEM((K, H), jnp.float32),
        pltpu.VMEM((K, H), jnp.float32),
        pltpu.VMEM((ZROWS, H), jnp.float32),
        pltpu.VMEM_SHARED((N, H), jnp.float32),
        pltpu.SemaphoreType.DMA,
        pltpu.SemaphoreType.DMA,
        pltpu.SemaphoreType.DMA,
        pltpu.SemaphoreType.DMA,
        pltpu.SemaphoreType.DMA,
        pltpu.SemaphoreType.DMA,
        pltpu.SemaphoreType.DMA,
        pltpu.SemaphoreType.DMA,
    ],
    compiler_params=pltpu.CompilerParams(needs_layout_passes=False, use_tc_tiling_on_sc=False),
)
def _agg_kernel(epk_hbm, ys_hbm, parts_hbm,
                ib0, ib1, ib2, ib3, rows0, rows1,
                zbuf_v, acc_sh,
                isem0, isem1, isem2, isem3, gsem0, gsem1, ssem0, ssem1):
    c = lax.axis_index("c")
    s = lax.axis_index("s")
    wid = s * NC + c

    ib = (ib0, ib1, ib2, ib3)
    rows = (rows0, rows1)
    isem = (isem0, isem1, isem2, isem3)
    gsem = (gsem0, gsem1)
    ssem = (ssem0, ssem1)

    def zero(i, carry):
        for v in range(H // 16):
            zbuf_v[i, pl.ds(v * 16, 16)] = jnp.zeros((16,), jnp.float32)
        return carry

    lax.fori_loop(0, ZROWS, zero, 0)
    for k in range(RP_S // ZROWS):
        pltpu.sync_copy(zbuf_v, acc_sh.at[pl.ds(s * RP_S + k * ZROWS, ZROWS)])
    plsc.subcore_barrier()

    ibase = wid * NCHUNK

    def idx_start(cix, q):
        pltpu.async_copy(epk_hbm.at[ibase + cix], ib[q], isem[q])

    def idx_wait(q):
        pltpu.make_async_copy(epk_hbm.at[0], ib[q], isem[q]).wait()

    def gather_start(b, q):
        pltpu.async_copy(ys_hbm.at[ib[q].at[0]], rows[b], gsem[b])

    def gather_wait(b, q):
        pltpu.make_async_copy(ys_hbm.at[ib[q].at[0]], rows[b], gsem[b]).wait()

    def scale(b, q):
        @plsc.parallel_loop(0, K, unroll=8)
        def body(j):
            idx16 = jnp.zeros((16,), jnp.int32) + j
            two16 = jnp.full((16,), 2, jnp.int32)
            raw = plsc.load_gather(ib[q], [two16, idx16])
            bc = plsc.bitcast(raw, jnp.float32)
            for v in range(H // 16):
                rows[b][j, pl.ds(v * 16, 16)] = \
                    rows[b][j, pl.ds(v * 16, 16)] * bc

    def scatter_start(b, q):
        pltpu.async_copy(rows[b], acc_sh.at[ib[q].at[1]], ssem[b], add=True)

    def scatter_wait(b, q):
        pltpu.make_async_copy(rows[b], acc_sh.at[ib[q].at[1]],
                              ssem[b]).wait()

    idx_start(0, 0)
    idx_wait(0)
    gather_start(0, 0)
    idx_start(1, 1)
    idx_wait(1)
    gather_start(1, 1)
    idx_start(2, 2)
    gather_wait(0, 0)
    scale(0, 0)
    scatter_start(0, 0)

    def grp(kk, carry):
        base = 1 + 4 * kk
        for i, (b, q) in enumerate(((1, 1), (0, 2), (1, 3), (0, 0))):
            cix = base + i
            q1 = (q + 1) % 4
            q2 = (q + 2) % 4

            @pl.when(cix + 1 < NCHUNK)
            def _():
                idx_wait(q1)

            scatter_wait(1 - b, (q - 1) % 4)

            @pl.when(cix + 1 < NCHUNK)
            def _():
                gather_start(1 - b, q1)

            @pl.when(cix + 2 < NCHUNK)
            def _():
                idx_start(cix + 2, q2)

            gather_wait(b, q)
            scale(b, q)
            scatter_start(b, q)
        return carry

    lax.fori_loop(0, (NCHUNK - 1) // 4, grp, 0)
    scatter_wait(0, 0)

    plsc.subcore_barrier()
    for k in range(RP_S // ZROWS):
        off = s * RP_S + k * ZROWS
        pltpu.sync_copy(acc_sh.at[pl.ds(off, ZROWS)],
                        parts_hbm.at[c, pl.ds(off, ZROWS)])


def _tcA_body(x_ref, w1_ref, degs_ref, ys_ref, dinv_ref):
    deg = jnp.sum(degs_ref[...], axis=0) + 1.0
    dinv = lax.rsqrt(deg)
    xw = lax.dot_general(x_ref[...], w1_ref[...], (((1,), (1,)), ((), ())),
                         preferred_element_type=jnp.float32)
    ys_ref[...] = xw * dinv[:, None]
    dinv_ref[...] = dinv


def _tcA(x, W1, degs):
    return pl.pallas_call(
        _tcA_body,
        out_shape=[jax.ShapeDtypeStruct((N, H), jnp.float32),
                   jax.ShapeDtypeStruct((N,), jnp.float32)],
    )(x, W1, degs)


def _tcC_body(parts_ref, ys1_ref, dinv_ref, b1_ref, w2_ref, ys2_ref):
    dinv = dinv_ref[...]
    acc = parts_ref[0] + parts_ref[1] + ys1_ref[...]
    h1 = jnp.maximum(acc * dinv[:, None] + b1_ref[...][None, :], 0.0)
    xw2 = lax.dot_general(h1, w2_ref[...], (((1,), (1,)), ((), ())),
                          preferred_element_type=jnp.float32)
    ys2_ref[...] = xw2 * dinv[:, None]


def _tcC(parts1, ys1, dinv, b1, W2):
    return pl.pallas_call(
        _tcC_body,
        out_shape=jax.ShapeDtypeStruct((N, H), jnp.float32),
    )(parts1, ys1, dinv, b1, W2)


def _tcD_body(parts_ref, ys2_ref, dinv_ref, b2_ref, batch_ref, emb_ref,
              wah_ref, wae_ref, ba_ref, ws_ref, bs_ref, wo_ref, bo_ref,
              out_ref):
    dinv = dinv_ref[...]
    acc = parts_ref[0] + parts_ref[1] + ys2_ref[...]
    h2 = jnp.maximum(acc * dinv[:, None] + b2_ref[...][None, :], 0.0)

    P = lax.dot_general(emb_ref[...], wae_ref[...], (((1,), (1,)), ((), ())),
                        preferred_element_type=jnp.float32)
    batch = batch_ref[...]
    oh = (batch[:, None] == lax.broadcasted_iota(jnp.int32, (1, B), 1)
          ).astype(jnp.float32)
    embp = lax.dot_general(oh, P, (((1,), (0,)), ((), ())),
                           preferred_element_type=jnp.float32)
    ah = lax.dot_general(h2, wah_ref[...], (((1,), (1,)), ((), ())),
                         preferred_element_type=jnp.float32) + embp \
        + ba_ref[...][None, :]
    sc = jnp.where(ah >= 0, ah, 0.2 * ah)
    logits = jnp.sum(sc * ws_ref[...], axis=1, keepdims=True) \
        + bs_ref[...][None, :]
    m = jnp.max(logits)
    e = jnp.exp(logits - m)
    aw = e / jnp.sum(e)
    w = aw * h2
    S = lax.dot_general(oh, w, (((0,), (0,)), ((), ())),
                        preferred_element_type=jnp.float32)
    cnt = jnp.sum(oh, axis=0)
    num = jnp.sum(S * wo_ref[...][0][None, :], axis=1) + cnt * bo_ref[...]
    out_ref[...] = num / jnp.maximum(cnt, 1.0)


def _tcD(parts2, ys2, dinv, b2, batch, app_embedding, Wah, Wae, ba, Ws, bs,
         Wo, bo):
    return pl.pallas_call(
        _tcD_body,
        out_shape=jax.ShapeDtypeStruct((B,), jnp.float32),
    )(parts2, ys2, dinv, b2, batch, app_embedding, Wah, Wae, ba, Ws, bs,
      Wo, bo)


def kernel(x, edge_index, edge_attr, batch, app_embedding, W1, b1, W2, b2,
           Wa, ba, Ws, bs, Wo, bo):
    src = edge_index[0]
    dst = edge_index[1]
    ewi = lax.bitcast_convert_type(edge_attr, jnp.int32)
    epk = jnp.stack([src, dst, ewi], 0).reshape(3, NW * NCHUNK, K)
    epk = epk.transpose(1, 0, 2)
    degs = _deg_kernel(dst, edge_attr)
    ys1, dinv = _tcA(x, W1, degs)
    parts1 = _agg_kernel(epk, ys1)
    ys2 = _tcC(parts1, ys1, dinv, b1, W2)
    parts2 = _agg_kernel(epk, ys2)
    Wah = Wa[:, :H]
    Wae = Wa[:, H:]
    return _tcD(parts2, ys2, dinv, b2, batch, app_embedding, Wah, Wae,
                ba, Ws, bs, Wo, bo)


# speedup vs baseline: 29.2339x; 1.1137x over previous
import functools

import jax
import jax.numpy as jnp
from jax import lax
from jax.experimental import pallas as pl
from jax.experimental.pallas import tpu as pltpu
from jax.experimental.pallas import tpu_sc as plsc

N = 10000
E = 320000
B = 4
D = 128
H = 128
EMB = 64
ATT = 64

NC = 2
NS = 16
NW = NC * NS
EP_W = E // NW
K = 80
NCHUNK = EP_W // K
ZROWS = 125
RP_S = N // NS

_mesh = plsc.VectorSubcoreMesh(core_axis_name="c", subcore_axis_name="s")


@functools.partial(
    pl.kernel,
    out_type=jax.ShapeDtypeStruct((NW, N), jnp.float32),
    mesh=_mesh,
    scratch_types=[
        pltpu.VMEM((EP_W,), jnp.int32),
        pltpu.VMEM((EP_W,), jnp.float32),
        pltpu.VMEM((N,), jnp.float32),
    ],
    compiler_params=pltpu.CompilerParams(needs_layout_passes=False, use_tc_tiling_on_sc=False),
)
def _deg_kernel(dst_hbm, ew_hbm, degs_hbm, dst_v, ew_v, deg_v):
    wid = lax.axis_index("s") * NC + lax.axis_index("c")
    base = wid * EP_W

    def zero(i, carry):
        deg_v[pl.ds(i * 16, 16)] = jnp.zeros((16,), jnp.float32)
        return carry

    lax.fori_loop(0, N // 16, zero, 0)

    pltpu.sync_copy(dst_hbm.at[pl.ds(base, EP_W)], dst_v)
    pltpu.sync_copy(ew_hbm.at[pl.ds(base, EP_W)], ew_v)

    def body(j, carry):
        d16 = dst_v[pl.ds(j * 16, 16)]
        w16 = ew_v[pl.ds(j * 16, 16)]
        plsc.addupdate_scatter(deg_v, [d16], w16)
        return carry

    lax.fori_loop(0, EP_W // 16, body, 0)
    pltpu.sync_copy(deg_v, degs_hbm.at[wid])


@functools.partial(
    pl.kernel,
    out_type=jax.ShapeDtypeStruct((NC, N, H), jnp.float32),
    mesh=_mesh,
    scratch_types=(
        [pltpu.VMEM((3, K), jnp.int32)] * 8
        + [pltpu.VMEM((K, H), jnp.float32)] * 4
        + [pltpu.VMEM_SHARED((N, H), jnp.float32)]
        + [pltpu.SemaphoreType.DMA] * 16
    ),
    compiler_params=pltpu.CompilerParams(needs_layout_passes=False, use_tc_tiling_on_sc=False),
)
def _agg_kernel(epk_hbm, ys_hbm, parts_hbm, *refs):
    ib = refs[0:8]
    rows = refs[8:12]
    acc_sh = refs[12]
    isem = refs[13:21]
    gsem = refs[21:25]
    ssem = refs[25:29]

    c = lax.axis_index("c")
    s = lax.axis_index("s")
    wid = s * NC + c
    ibase = wid * NCHUNK

    def zero(i, carry):
        for v in range(H // 16):
            rows[0][i, pl.ds(v * 16, 16)] = jnp.zeros((16,), jnp.float32)
        return carry

    lax.fori_loop(0, K, zero, 0)
    for k in range(RP_S // K):
        pltpu.sync_copy(rows[0], acc_sh.at[pl.ds(s * RP_S + k * K, K)])
    rem = RP_S % K
    if rem:
        pltpu.sync_copy(rows[0].at[pl.ds(0, rem)],
                        acc_sh.at[pl.ds(s * RP_S + (RP_S // K) * K, rem)])
    plsc.subcore_barrier()

    def idx_start(cix, p):
        pltpu.async_copy(epk_hbm.at[ibase + cix], ib[p], isem[p])

    def idx_wait(p):
        pltpu.make_async_copy(epk_hbm.at[0], ib[p], isem[p]).wait()

    def gather_start(p, b):
        pltpu.async_copy(ys_hbm.at[ib[p].at[0]], rows[b], gsem[b])

    def gather_wait(p, b):
        pltpu.make_async_copy(ys_hbm.at[ib[p].at[0]], rows[b],
                              gsem[b]).wait()

    def scale(p, b):
        two16 = jnp.full((16,), 2, jnp.int32)

        @plsc.parallel_loop(0, K, unroll=8)
        def body(j):
            idx16 = jnp.zeros((16,), jnp.int32) + j
            raw = plsc.load_gather(ib[p], [two16, idx16])
            bc = plsc.bitcast(raw, jnp.float32)
            for v in range(H // 16):
                rows[b][j, pl.ds(v * 16, 16)] = \
                    rows[b][j, pl.ds(v * 16, 16)] * bc

    def scatter_start(p, b):
        pltpu.async_copy(rows[b], acc_sh.at[ib[p].at[1]], ssem[b], add=True)

    def scatter_wait(p, b):
        pltpu.make_async_copy(rows[b], acc_sh.at[ib[p].at[1]],
                              ssem[b]).wait()

    def step(cix, m, first=False, last=False, more_idx=True):
        p, b = m % 8, m % 4
        p2, b2 = (m + 2) % 8, (m + 2) % 4
        if not first:
            scatter_wait((m - 2) % 8, (m - 2) % 4)
        if not last:
            idx_wait(p2)
            gather_start(p2, b2)
        if more_idx:
            idx_start(cix + 5, (m + 5) % 8)
        gather_wait(p, b)
        scale(p, b)
        scatter_start(p, b)

    for cix in range(5):
        idx_start(cix, cix)
    idx_wait(0)
    gather_start(0, 0)
    idx_wait(1)
    gather_start(1, 1)
    step(0, 0, first=True)
    step(1, 1, first=True)
    step(2, 2)
    step(3, 3)
    step(4, 4)

    def grp(kk, carry):
        base = 5 + 8 * kk
        for i in range(8):
            step(base + i, 5 + i)
        return carry

    lax.fori_loop(0, (NCHUNK - 5) // 8 - 1, grp, 0)
    for cix in range(NCHUNK - 8, NCHUNK):
        step(cix, cix, last=(cix + 2 >= NCHUNK),
             more_idx=(cix + 5 < NCHUNK))
    scatter_wait((NCHUNK - 2) % 8, (NCHUNK - 2) % 4)
    scatter_wait((NCHUNK - 1) % 8, (NCHUNK - 1) % 4)

    plsc.subcore_barrier()
    for k in range(RP_S // ZROWS):
        off = s * RP_S + k * ZROWS
        pltpu.sync_copy(acc_sh.at[pl.ds(off, ZROWS)],
                        parts_hbm.at[c, pl.ds(off, ZROWS)])


def _tcA_body(x_ref, w1_ref, degs_ref, ys_ref, dinv_ref):
    deg = jnp.sum(degs_ref[...], axis=0) + 1.0
    dinv = lax.rsqrt(deg)
    xw = lax.dot_general(x_ref[...], w1_ref[...], (((1,), (1,)), ((), ())),
                         preferred_element_type=jnp.float32)
    ys_ref[...] = xw * dinv[:, None]
    dinv_ref[...] = dinv


def _tcA(x, W1, degs):
    return pl.pallas_call(
        _tcA_body,
        out_shape=[jax.ShapeDtypeStruct((N, H), jnp.float32),
                   jax.ShapeDtypeStruct((N,), jnp.float32)],
    )(x, W1, degs)


def _tcC_body(parts_ref, ys1_ref, dinv_ref, b1_ref, w2_ref, ys2_ref):
    dinv = dinv_ref[...]
    acc = parts_ref[0] + parts_ref[1] + ys1_ref[...]
    h1 = jnp.maximum(acc * dinv[:, None] + b1_ref[...][None, :], 0.0)
    xw2 = lax.dot_general(h1, w2_ref[...], (((1,), (1,)), ((), ())),
                          preferred_element_type=jnp.float32)
    ys2_ref[...] = xw2 * dinv[:, None]


def _tcC(parts1, ys1, dinv, b1, W2):
    return pl.pallas_call(
        _tcC_body,
        out_shape=jax.ShapeDtypeStruct((N, H), jnp.float32),
    )(parts1, ys1, dinv, b1, W2)


def _tcD_body(parts_ref, ys2_ref, dinv_ref, b2_ref, batch_ref, emb_ref,
              wah_ref, wae_ref, ba_ref, ws_ref, bs_ref, wo_ref, bo_ref,
              out_ref):
    dinv = dinv_ref[...]
    acc = parts_ref[0] + parts_ref[1] + ys2_ref[...]
    h2 = jnp.maximum(acc * dinv[:, None] + b2_ref[...][None, :], 0.0)

    P = lax.dot_general(emb_ref[...], wae_ref[...], (((1,), (1,)), ((), ())),
                        preferred_element_type=jnp.float32)
    batch = batch_ref[...]
    oh = (batch[:, None] == lax.broadcasted_iota(jnp.int32, (1, B), 1)
          ).astype(jnp.float32)
    embp = lax.dot_general(oh, P, (((1,), (0,)), ((), ())),
                           preferred_element_type=jnp.float32)
    ah = lax.dot_general(h2, wah_ref[...], (((1,), (1,)), ((), ())),
                         preferred_element_type=jnp.float32) + embp \
        + ba_ref[...][None, :]
    sc = jnp.where(ah >= 0, ah, 0.2 * ah)
    logits = jnp.sum(sc * ws_ref[...], axis=1, keepdims=True) \
        + bs_ref[...][None, :]
    m = jnp.max(logits)
    e = jnp.exp(logits - m)
    aw = e / jnp.sum(e)
    w = aw * h2
    S = lax.dot_general(oh, w, (((0,), (0,)), ((), ())),
                        preferred_element_type=jnp.float32)
    cnt = jnp.sum(oh, axis=0)
    num = jnp.sum(S * wo_ref[...][0][None, :], axis=1) + cnt * bo_ref[...]
    out_ref[...] = num / jnp.maximum(cnt, 1.0)


def _tcD(parts2, ys2, dinv, b2, batch, app_embedding, Wah, Wae, ba, Ws, bs,
         Wo, bo):
    return pl.pallas_call(
        _tcD_body,
        out_shape=jax.ShapeDtypeStruct((B,), jnp.float32),
    )(parts2, ys2, dinv, b2, batch, app_embedding, Wah, Wae, ba, Ws, bs,
      Wo, bo)


def kernel(x, edge_index, edge_attr, batch, app_embedding, W1, b1, W2, b2,
           Wa, ba, Ws, bs, Wo, bo):
    src = edge_index[0]
    dst = edge_index[1]
    ewi = lax.bitcast_convert_type(edge_attr, jnp.int32)
    epk = jnp.stack([src, dst, ewi], 0).reshape(3, NW * NCHUNK, K)
    epk = epk.transpose(1, 0, 2)
    degs = _deg_kernel(dst, edge_attr)
    ys1, dinv = _tcA(x, W1, degs)
    parts1 = _agg_kernel(epk, ys1)
    ys2 = _tcC(parts1, ys1, dinv, b1, W2)
    parts2 = _agg_kernel(epk, ys2)
    Wah = Wa[:, :H]
    Wae = Wa[:, H:]
    return _tcD(parts2, ys2, dinv, b2, batch, app_embedding, Wah, Wae,
                ba, Ws, bs, Wo, bo)
